# baseline (speedup 1.0000x reference)
import jax
import jax.numpy as jnp
from jax import lax
from jax.experimental import pallas as pl
from jax.experimental.pallas import tpu as pltpu
from jax.experimental.pallas import tpu_sc as plsc

N = 10000
H = 32
CIN = 128
COUT = 128
CMID = 32
K = 15
CPG = 16
HID = 8
SIGMA = 1.0
LEAK = 0.1
EPS = 1e-5

TBW = 64
TW = 128
HQ = H // 4

SC_NC = 2
SC_NS = 16
NW = SC_NC * SC_NS
BPW = N * H // NW
CB = 400
NCH = BPW // CB


def _leaky(x):
    return jnp.where(x >= 0, x, LEAK * x)


def _bn_affine(st_ref, g_ref, b_ref):
    mean = st_ref[0:1, :] * (1.0 / N)
    var = st_ref[1:2, :] * (1.0 / N) - mean * mean
    scale = g_ref[...] / jnp.sqrt(var + EPS)
    bias = b_ref[...] - mean * scale
    return scale, bias


def _acc_stats(st_ref, v, width):
    s = jnp.sum(v, axis=0, keepdims=True)
    sq = jnp.sum(v * v, axis=0, keepdims=True)
    blk = jnp.concatenate([s, sq, jnp.zeros((6, width), jnp.float32)], axis=0)

    @pl.when(pl.program_id(0) == 0)
    def _():
        st_ref[...] = jnp.zeros_like(st_ref)

    st_ref[...] += blk


def _k1(x_ref, w_ref, z_ref, st_ref):
    z = jnp.dot(x_ref[...], w_ref[...], preferred_element_type=jnp.float32)
    z_ref[...] = z
    _acc_stats(st_ref, z, CMID)


def _k2(z_ref, p_ref, st_ref, g_ref, b_ref, t_ref):
    scale, bias = _bn_affine(st_ref, g_ref, b_ref)
    x = _leaky(z_ref[...] * scale + bias)
    bm = z_ref.shape[0]
    t_ref[...] = jnp.concatenate(
        [x, p_ref[...], jnp.zeros((bm, 29), jnp.float32)], axis=1)


def _sc_gather(tbl_hbm, idx_hbm, xg_hbm, pr_hbm,
               idx_v, buf0, buf1, sg0, sg1, sw0, sw1):
    wid = lax.axis_index("s") * SC_NC + lax.axis_index("c")
    pltpu.sync_copy(idx_hbm.at[wid], idx_v)

    def wrts(c, buf, sem):
        base = wid * BPW + c * CB
        pltpu.async_copy(buf.at[:, 0:32], xg_hbm.at[pl.ds(base, CB)], sem)
        pltpu.async_copy(buf.at[:, 32:64], pr_hbm.at[pl.ds(base, CB)], sem)

    def drain(buf, sem):
        pltpu.make_async_copy(tbl_hbm.at[pl.ds(0, CB)], buf, sem).wait()

    pltpu.async_copy(tbl_hbm.at[idx_v.at[0]], buf0, sg0)

    def body(t, _):
        a = 2 * t + 1
        b = 2 * t + 2
        pltpu.async_copy(tbl_hbm.at[idx_v.at[a]], buf1, sg1)
        drain(buf0, sg0)
        wrts(2 * t, buf0, sw0)
        drain(buf0, sw0)
        pltpu.async_copy(tbl_hbm.at[idx_v.at[b]], buf0, sg0)
        drain(buf1, sg1)
        wrts(a, buf1, sw1)
        drain(buf1, sw1)
        return 0

    lax.fori_loop(0, (NCH - 1) // 2, body, 0)
    drain(buf0, sg0)
    wrts(NCH - 1, buf0, sw0)
    drain(buf0, sw0)


def _k3(xg_ref, pr_ref, q_ref, kpn2_ref, m1_ref, m2_ref, wg1_ref, bg1_ref,
        wg2e_ref, bg2e_ref, wg2o_ref, bg2o_ref, sa_ref, s2_ref,
        out_ref, st_ref):
    bm = out_ref.shape[0]
    xg = xg_ref[...].reshape(bm, HQ, TW)
    qpad = jnp.concatenate(
        [q_ref[...], jnp.zeros((bm, 29), jnp.float32)], axis=1)
    qrep = jnp.tile(qpad, (1, HQ // 2))
    up = pr_ref[...].reshape(bm, HQ, TW) - qrep[:, None, :]
    ukp2 = jnp.dot(up.reshape(bm * HQ, TW), m1_ref[...],
                   preferred_element_type=jnp.float32)
    usq = up * up
    urep = jnp.dot(usq.reshape(bm * HQ, TW), m2_ref[...],
                   preferred_element_type=jnp.float32)
    sq = jnp.maximum(urep + ukp2 + kpn2_ref[...], 0.0)
    infl = jnp.maximum(1.0 - jnp.sqrt(sq) * (1.0 / SIGMA),
                       0.0).reshape(bm, HQ, TW)

    center = xg[:, 0, 0:CMID]
    hmid = _leaky(jnp.dot(center, wg1_ref[...],
                          preferred_element_type=jnp.float32) + bg1_ref[...])
    wv0 = jnp.dot(hmid, wg2e_ref[...],
                  preferred_element_type=jnp.float32) + bg2e_ref[...]
    wv1 = jnp.dot(hmid, wg2o_ref[...],
                  preferred_element_type=jnp.float32) + bg2o_ref[...]
    wvt = jnp.tile(jnp.concatenate([wv0, wv1], axis=1), (1, 4))

    aw = infl * wvt[:, None, :]
    a = jnp.dot(aw.reshape(bm * HQ, TW), sa_ref[...],
                preferred_element_type=jnp.float32).reshape(bm, HQ, TW)
    zs = jnp.sum(xg * a, axis=1)
    outv = jnp.dot(zs, s2_ref[...], preferred_element_type=jnp.float32)
    out_ref[...] = outv
    _acc_stats(st_ref, outv, CMID)


def _k4(y_ref, st2_ref, gk_ref, bk_ref, w2_ref, t_ref, st3_ref):
    scale, bias = _bn_affine(st2_ref, gk_ref, bk_ref)
    l = _leaky(y_ref[...] * scale + bias)
    t = jnp.dot(l, w2_ref[...], preferred_element_type=jnp.float32)
    t_ref[...] = t
    _acc_stats(st3_ref, t, COUT)


def _k5(t_ref, st3_ref, g2_ref, b2_ref, sf_ref, o_ref):
    scale, bias = _bn_affine(st3_ref, g2_ref, b2_ref)
    o_ref[...] = _leaky(t_ref[...] * scale + bias + sf_ref[...])


def kernel(q_pts, s_pts, s_feats, neighbor_indices, kernel_points,
           W1, g1, b1, Wg1, bg1, Wg2, bg2, gk, bk, W2, g2, b2):
    f32 = jnp.float32
    BM1 = 2000
    z, st1 = pl.pallas_call(
        _k1,
        grid=(N // BM1,),
        in_specs=[pl.BlockSpec((BM1, CIN), lambda i: (i, 0)),
                  pl.BlockSpec((CIN, CMID), lambda i: (0, 0))],
        out_specs=[pl.BlockSpec((BM1, CMID), lambda i: (i, 0)),
                   pl.BlockSpec((8, CMID), lambda i: (0, 0))],
        out_shape=[jax.ShapeDtypeStruct((N, CMID), f32),
                   jax.ShapeDtypeStruct((8, CMID), f32)],
    )(s_feats, W1)

    tbl = pl.pallas_call(
        _k2,
        grid=(N // BM1,),
        in_specs=[pl.BlockSpec((BM1, CMID), lambda i: (i, 0)),
                  pl.BlockSpec((BM1, 3), lambda i: (i, 0)),
                  pl.BlockSpec((8, CMID), lambda i: (0, 0)),
                  pl.BlockSpec((1, CMID), lambda i: (0, 0)),
                  pl.BlockSpec((1, CMID), lambda i: (0, 0))],
        out_specs=pl.BlockSpec((BM1, TBW), lambda i: (i, 0)),
        out_shape=jax.ShapeDtypeStruct((N, TBW), f32),
    )(z, s_pts, st1, g1.reshape(1, CMID), b1.reshape(1, CMID))

    gather = pl.kernel(
        _sc_gather,
        out_type=[jax.ShapeDtypeStruct((N * H, 32), f32),
                  jax.ShapeDtypeStruct((N * H, 32), f32)],
        scratch_types=[pltpu.VMEM((NCH, CB), jnp.int32),
                       pltpu.VMEM((CB, TBW), f32),
                       pltpu.VMEM((CB, TBW), f32),
                       pltpu.SemaphoreType.DMA,
                       pltpu.SemaphoreType.DMA,
                       pltpu.SemaphoreType.DMA,
                       pltpu.SemaphoreType.DMA],
        mesh=plsc.VectorSubcoreMesh(core_axis_name="c", subcore_axis_name="s"),
        compiler_params=pltpu.CompilerParams(use_tc_tiling_on_sc=False),
    )
    xg_o, pr_o = gather(tbl, neighbor_indices.reshape(NW, NCH, CB))
    xg = xg_o.reshape(N * H // 4, TW)
    prg = pr_o.reshape(N * H // 4, TW)
    return xg[0:N, 0:128] + prg[0:N, 0:128]

    kpad = jnp.concatenate(
        [kernel_points.T, jnp.full((3, 1), 1e6, f32)], axis=1)
    kpb = jnp.concatenate(
        [jnp.tile(kpad, (1, 2)), jnp.zeros((29, 32), f32)], axis=0)
    m1 = jnp.kron(jnp.eye(4, dtype=f32), -2.0 * kpb)
    m2 = jnp.kron(jnp.eye(4, dtype=f32), jnp.ones((32, 32), f32))
    kpn2 = jnp.tile(jnp.sum(kpad * kpad, axis=0).reshape(1, 16), (1, 8))
    Wg2r = jnp.concatenate(
        [Wg2.reshape(HID, K, 2), jnp.zeros((HID, 1, 2), f32)], axis=1)
    bg2r = jnp.concatenate([bg2.reshape(K, 2), jnp.zeros((1, 2), f32)], axis=0)
    sa = jnp.kron(jnp.eye(8, dtype=f32), jnp.ones((16, 16), f32))
    s2 = jnp.kron(jnp.ones((4, 1), f32), jnp.eye(32, dtype=f32))

    BM3 = 1000
    out, st2 = pl.pallas_call(
        _k3,
        grid=(N // BM3,),
        in_specs=[pl.BlockSpec((BM3 * HQ, TW), lambda i: (i, 0)),
                  pl.BlockSpec((BM3 * HQ, TW), lambda i: (i, 0)),
                  pl.BlockSpec((BM3, 3), lambda i: (i, 0)),
                  pl.BlockSpec((1, TW), lambda i: (0, 0)),
                  pl.BlockSpec((TW, TW), lambda i: (0, 0)),
                  pl.BlockSpec((TW, TW), lambda i: (0, 0)),
                  pl.BlockSpec((CMID, HID), lambda i: (0, 0)),
                  pl.BlockSpec((1, HID), lambda i: (0, 0)),
                  pl.BlockSpec((HID, 16), lambda i: (0, 0)),
                  pl.BlockSpec((1, 16), lambda i: (0, 0)),
                  pl.BlockSpec((HID, 16), lambda i: (0, 0)),
                  pl.BlockSpec((1, 16), lambda i: (0, 0)),
                  pl.BlockSpec((TW, TW), lambda i: (0, 0)),
                  pl.BlockSpec((TW, CMID), lambda i: (0, 0))],
        out_specs=[pl.BlockSpec((BM3, CMID), lambda i: (i, 0)),
                   pl.BlockSpec((8, CMID), lambda i: (0, 0))],
        out_shape=[jax.ShapeDtypeStruct((N, CMID), f32),
                   jax.ShapeDtypeStruct((8, CMID), f32)],
    )(xg, prg, q_pts, kpn2, m1, m2, Wg1, bg1.reshape(1, HID),
      Wg2r[:, :, 0], bg2r[:, 0].reshape(1, 16),
      Wg2r[:, :, 1], bg2r[:, 1].reshape(1, 16), sa, s2)

    BM4 = 2000
    t2, st3 = pl.pallas_call(
        _k4,
        grid=(N // BM4,),
        in_specs=[pl.BlockSpec((BM4, CMID), lambda i: (i, 0)),
                  pl.BlockSpec((8, CMID), lambda i: (0, 0)),
                  pl.BlockSpec((1, CMID), lambda i: (0, 0)),
                  pl.BlockSpec((1, CMID), lambda i: (0, 0)),
                  pl.BlockSpec((CMID, COUT), lambda i: (0, 0))],
        out_specs=[pl.BlockSpec((BM4, COUT), lambda i: (i, 0)),
                   pl.BlockSpec((8, COUT), lambda i: (0, 0))],
        out_shape=[jax.ShapeDtypeStruct((N, COUT), f32),
                   jax.ShapeDtypeStruct((8, COUT), f32)],
    )(out, st2, gk.reshape(1, CMID), bk.reshape(1, CMID), W2)

    res = pl.pallas_call(
        _k5,
        grid=(N // BM4,),
        in_specs=[pl.BlockSpec((BM4, COUT), lambda i: (i, 0)),
                  pl.BlockSpec((8, COUT), lambda i: (0, 0)),
                  pl.BlockSpec((1, COUT), lambda i: (0, 0)),
                  pl.BlockSpec((1, COUT), lambda i: (0, 0)),
                  pl.BlockSpec((BM4, COUT), lambda i: (i, 0))],
        out_specs=pl.BlockSpec((BM4, COUT), lambda i: (i, 0)),
        out_shape=jax.ShapeDtypeStruct((N, COUT), f32),
    )(t2, st3, g2.reshape(1, COUT), b2.reshape(1, COUT), s_feats)

    return res


# speedup vs baseline: 1.3774x; 1.3774x over previous
import jax
import jax.numpy as jnp
from jax import lax
from jax.experimental import pallas as pl
from jax.experimental.pallas import tpu as pltpu
from jax.experimental.pallas import tpu_sc as plsc

N = 10000
H = 32
CIN = 128
COUT = 128
CMID = 32
K = 15
CPG = 16
HID = 8
SIGMA = 1.0
LEAK = 0.1
EPS = 1e-5

TBW = 64
TW = 128
HQ = H // 4

SC_NC = 2
SC_NS = 16
NW = SC_NC * SC_NS
BPW = N * H // NW
CB = 400
NCH = BPW // CB


def _leaky(x):
    return jnp.where(x >= 0, x, LEAK * x)


def _bn_affine(st_ref, g_ref, b_ref):
    mean = st_ref[0:1, :] * (1.0 / N)
    var = st_ref[1:2, :] * (1.0 / N) - mean * mean
    scale = g_ref[...] / jnp.sqrt(var + EPS)
    bias = b_ref[...] - mean * scale
    return scale, bias


def _acc_stats(st_ref, v, width):
    s = jnp.sum(v, axis=0, keepdims=True)
    sq = jnp.sum(v * v, axis=0, keepdims=True)
    blk = jnp.concatenate([s, sq, jnp.zeros((6, width), jnp.float32)], axis=0)

    @pl.when(pl.program_id(0) == 0)
    def _():
        st_ref[...] = jnp.zeros_like(st_ref)

    st_ref[...] += blk


def _k1(x_ref, w_ref, z_ref, st_ref):
    z = jnp.dot(x_ref[...], w_ref[...], preferred_element_type=jnp.float32)
    z_ref[...] = z
    _acc_stats(st_ref, z, CMID)


def _k2(z_ref, p_ref, st_ref, g_ref, b_ref, t_ref):
    scale, bias = _bn_affine(st_ref, g_ref, b_ref)
    x = _leaky(z_ref[...] * scale + bias)
    bm = z_ref.shape[0]
    t_ref[...] = jnp.concatenate(
        [x, p_ref[...], jnp.zeros((bm, 29), jnp.float32)], axis=1)


def _sc_gather(tbl_hbm, idx_hbm, xg_hbm, pr_hbm,
               idx_v, buf0, buf1, sg0, sg1, sw0, sw1):
    wid = lax.axis_index("s") * SC_NC + lax.axis_index("c")
    pltpu.sync_copy(idx_hbm.at[wid], idx_v)

    def wr_descs(c, buf, sem):
        base = wid * (BPW // 4) + c * (CB // 4)
        ds = []
        for s in range(4):
            ds.append(pltpu.make_async_copy(
                buf.at[pl.ds(s * (CB // 4), CB // 4), 0:32],
                xg_hbm.at[pl.ds(base, CB // 4), pl.ds(s * 32, 32)], sem))
            ds.append(pltpu.make_async_copy(
                buf.at[pl.ds(s * (CB // 4), CB // 4), 32:64],
                pr_hbm.at[pl.ds(base, CB // 4), pl.ds(s * 32, 32)], sem))
        return ds

    def wrts(c, buf, sem):
        for d in wr_descs(c, buf, sem):
            d.start()

    def wdrain(c, buf, sem):
        for d in wr_descs(c, buf, sem):
            d.wait()

    def gdrain(c, buf, sem):
        pltpu.make_async_copy(tbl_hbm.at[idx_v.at[c]], buf, sem).wait()

    pltpu.async_copy(tbl_hbm.at[idx_v.at[0]], buf0, sg0)

    def body(t, _):
        a = 2 * t + 1
        b = 2 * t + 2
        pltpu.async_copy(tbl_hbm.at[idx_v.at[a]], buf1, sg1)
        gdrain(2 * t, buf0, sg0)
        wrts(2 * t, buf0, sw0)
        wdrain(2 * t, buf0, sw0)
        pltpu.async_copy(tbl_hbm.at[idx_v.at[b]], buf0, sg0)
        gdrain(a, buf1, sg1)
        wrts(a, buf1, sw1)
        wdrain(a, buf1, sw1)
        return 0

    lax.fori_loop(0, (NCH - 1) // 2, body, 0)
    gdrain(NCH - 1, buf0, sg0)
    wrts(NCH - 1, buf0, sw0)
    wdrain(NCH - 1, buf0, sw0)


def _k3(xg_ref, pr_ref, q_ref, kpn2_ref, m1_ref, m2_ref, wg1_ref, bg1_ref,
        wg2e_ref, bg2e_ref, wg2o_ref, bg2o_ref, sa_ref, s2_ref,
        out_ref, st_ref):
    bm = out_ref.shape[0]
    xg = xg_ref[...].reshape(bm, HQ, TW)
    qpad = jnp.concatenate(
        [q_ref[...], jnp.zeros((bm, 29), jnp.float32)], axis=1)
    qrep = jnp.tile(qpad, (1, HQ // 2))
    up = pr_ref[...].reshape(bm, HQ, TW) - qrep[:, None, :]
    ukp2 = jnp.dot(up.reshape(bm * HQ, TW), m1_ref[...],
                   preferred_element_type=jnp.float32)
    usq = up * up
    urep = jnp.dot(usq.reshape(bm * HQ, TW), m2_ref[...],
                   preferred_element_type=jnp.float32)
    sq = jnp.maximum(urep + ukp2 + kpn2_ref[...], 0.0)
    infl = jnp.maximum(1.0 - jnp.sqrt(sq) * (1.0 / SIGMA),
                       0.0).reshape(bm, HQ, TW)

    center = xg[:, 0, 0:CMID]
    hmid = _leaky(jnp.dot(center, wg1_ref[...],
                          preferred_element_type=jnp.float32) + bg1_ref[...])
    wv0 = jnp.dot(hmid, wg2e_ref[...],
                  preferred_element_type=jnp.float32) + bg2e_ref[...]
    wv1 = jnp.dot(hmid, wg2o_ref[...],
                  preferred_element_type=jnp.float32) + bg2o_ref[...]
    wvt = jnp.tile(jnp.concatenate([wv0, wv1], axis=1), (1, 4))

    aw = infl * wvt[:, None, :]
    a = jnp.dot(aw.reshape(bm * HQ, TW), sa_ref[...],
                preferred_element_type=jnp.float32).reshape(bm, HQ, TW)
    zs = jnp.sum(xg * a, axis=1)
    outv = jnp.dot(zs, s2_ref[...], preferred_element_type=jnp.float32)
    out_ref[...] = outv
    _acc_stats(st_ref, outv, CMID)


def _k4(y_ref, st2_ref, gk_ref, bk_ref, w2_ref, t_ref, st3_ref):
    scale, bias = _bn_affine(st2_ref, gk_ref, bk_ref)
    l = _leaky(y_ref[...] * scale + bias)
    t = jnp.dot(l, w2_ref[...], preferred_element_type=jnp.float32)
    t_ref[...] = t
    _acc_stats(st3_ref, t, COUT)


def _k5(t_ref, st3_ref, g2_ref, b2_ref, sf_ref, o_ref):
    scale, bias = _bn_affine(st3_ref, g2_ref, b2_ref)
    o_ref[...] = _leaky(t_ref[...] * scale + bias + sf_ref[...])


def kernel(q_pts, s_pts, s_feats, neighbor_indices, kernel_points,
           W1, g1, b1, Wg1, bg1, Wg2, bg2, gk, bk, W2, g2, b2):
    f32 = jnp.float32
    BM1 = 2000
    z, st1 = pl.pallas_call(
        _k1,
        grid=(N // BM1,),
        in_specs=[pl.BlockSpec((BM1, CIN), lambda i: (i, 0)),
                  pl.BlockSpec((CIN, CMID), lambda i: (0, 0))],
        out_specs=[pl.BlockSpec((BM1, CMID), lambda i: (i, 0)),
                   pl.BlockSpec((8, CMID), lambda i: (0, 0))],
        out_shape=[jax.ShapeDtypeStruct((N, CMID), f32),
                   jax.ShapeDtypeStruct((8, CMID), f32)],
    )(s_feats, W1)

    tbl = pl.pallas_call(
        _k2,
        grid=(N // BM1,),
        in_specs=[pl.BlockSpec((BM1, CMID), lambda i: (i, 0)),
                  pl.BlockSpec((BM1, 3), lambda i: (i, 0)),
                  pl.BlockSpec((8, CMID), lambda i: (0, 0)),
                  pl.BlockSpec((1, CMID), lambda i: (0, 0)),
                  pl.BlockSpec((1, CMID), lambda i: (0, 0))],
        out_specs=pl.BlockSpec((BM1, TBW), lambda i: (i, 0)),
        out_shape=jax.ShapeDtypeStruct((N, TBW), f32),
    )(z, s_pts, st1, g1.reshape(1, CMID), b1.reshape(1, CMID))

    gather = pl.kernel(
        _sc_gather,
        out_type=[jax.ShapeDtypeStruct((N * H // 4, TW), f32),
                  jax.ShapeDtypeStruct((N * H // 4, TW), f32)],
        scratch_types=[pltpu.VMEM((NCH, CB), jnp.int32),
                       pltpu.VMEM((CB, TBW), f32),
                       pltpu.VMEM((CB, TBW), f32),
                       pltpu.SemaphoreType.DMA,
                       pltpu.SemaphoreType.DMA,
                       pltpu.SemaphoreType.DMA,
                       pltpu.SemaphoreType.DMA],
        mesh=plsc.VectorSubcoreMesh(core_axis_name="c", subcore_axis_name="s"),
        compiler_params=pltpu.CompilerParams(use_tc_tiling_on_sc=False),
    )
    idxp = neighbor_indices.reshape(NW, NCH, CB // 4, 4).transpose(
        0, 1, 3, 2).reshape(NW, NCH, CB)
    xg, prg = gather(tbl, idxp)

    kpad = jnp.concatenate(
        [kernel_points.T, jnp.full((3, 1), 1e6, f32)], axis=1)
    kpb = jnp.concatenate(
        [jnp.tile(kpad, (1, 2)), jnp.zeros((29, 32), f32)], axis=0)
    m1 = jnp.kron(jnp.eye(4, dtype=f32), -2.0 * kpb)
    m2 = jnp.kron(jnp.eye(4, dtype=f32), jnp.ones((32, 32), f32))
    kpn2 = jnp.tile(jnp.sum(kpad * kpad, axis=0).reshape(1, 16), (1, 8))
    Wg2r = jnp.concatenate(
        [Wg2.reshape(HID, K, 2), jnp.zeros((HID, 1, 2), f32)], axis=1)
    bg2r = jnp.concatenate([bg2.reshape(K, 2), jnp.zeros((1, 2), f32)], axis=0)
    sa = jnp.kron(jnp.eye(8, dtype=f32), jnp.ones((16, 16), f32))
    s2 = jnp.kron(jnp.ones((4, 1), f32), jnp.eye(32, dtype=f32))

    BM3 = 1000
    out, st2 = pl.pallas_call(
        _k3,
        grid=(N // BM3,),
        in_specs=[pl.BlockSpec((BM3 * HQ, TW), lambda i: (i, 0)),
                  pl.BlockSpec((BM3 * HQ, TW), lambda i: (i, 0)),
                  pl.BlockSpec((BM3, 3), lambda i: (i, 0)),
                  pl.BlockSpec((1, TW), lambda i: (0, 0)),
                  pl.BlockSpec((TW, TW), lambda i: (0, 0)),
                  pl.BlockSpec((TW, TW), lambda i: (0, 0)),
                  pl.BlockSpec((CMID, HID), lambda i: (0, 0)),
                  pl.BlockSpec((1, HID), lambda i: (0, 0)),
                  pl.BlockSpec((HID, 16), lambda i: (0, 0)),
                  pl.BlockSpec((1, 16), lambda i: (0, 0)),
                  pl.BlockSpec((HID, 16), lambda i: (0, 0)),
                  pl.BlockSpec((1, 16), lambda i: (0, 0)),
                  pl.BlockSpec((TW, TW), lambda i: (0, 0)),
                  pl.BlockSpec((TW, CMID), lambda i: (0, 0))],
        out_specs=[pl.BlockSpec((BM3, CMID), lambda i: (i, 0)),
                   pl.BlockSpec((8, CMID), lambda i: (0, 0))],
        out_shape=[jax.ShapeDtypeStruct((N, CMID), f32),
                   jax.ShapeDtypeStruct((8, CMID), f32)],
    )(xg, prg, q_pts, kpn2, m1, m2, Wg1, bg1.reshape(1, HID),
      Wg2r[:, :, 0], bg2r[:, 0].reshape(1, 16),
      Wg2r[:, :, 1], bg2r[:, 1].reshape(1, 16), sa, s2)

    BM4 = 2000
    t2, st3 = pl.pallas_call(
        _k4,
        grid=(N // BM4,),
        in_specs=[pl.BlockSpec((BM4, CMID), lambda i: (i, 0)),
                  pl.BlockSpec((8, CMID), lambda i: (0, 0)),
                  pl.BlockSpec((1, CMID), lambda i: (0, 0)),
                  pl.BlockSpec((1, CMID), lambda i: (0, 0)),
                  pl.BlockSpec((CMID, COUT), lambda i: (0, 0))],
        out_specs=[pl.BlockSpec((BM4, COUT), lambda i: (i, 0)),
                   pl.BlockSpec((8, COUT), lambda i: (0, 0))],
        out_shape=[jax.ShapeDtypeStruct((N, COUT), f32),
                   jax.ShapeDtypeStruct((8, COUT), f32)],
    )(out, st2, gk.reshape(1, CMID), bk.reshape(1, CMID), W2)

    res = pl.pallas_call(
        _k5,
        grid=(N // BM4,),
        in_specs=[pl.BlockSpec((BM4, COUT), lambda i: (i, 0)),
                  pl.BlockSpec((8, COUT), lambda i: (0, 0)),
                  pl.BlockSpec((1, COUT), lambda i: (0, 0)),
                  pl.BlockSpec((1, COUT), lambda i: (0, 0)),
                  pl.BlockSpec((BM4, COUT), lambda i: (i, 0))],
        out_specs=pl.BlockSpec((BM4, COUT), lambda i: (i, 0)),
        out_shape=jax.ShapeDtypeStruct((N, COUT), f32),
    )(t2, st3, g2.reshape(1, COUT), b2.reshape(1, COUT), s_feats)

    return res


# speedup vs baseline: 1.7363x; 1.2605x over previous
import jax
import jax.numpy as jnp
from jax import lax
from jax.experimental import pallas as pl
from jax.experimental.pallas import tpu as pltpu
from jax.experimental.pallas import tpu_sc as plsc

N = 10000
H = 32
CIN = 128
COUT = 128
CMID = 32
K = 15
CPG = 16
HID = 8
SIGMA = 1.0
LEAK = 0.1
EPS = 1e-5

TBW = 64
TW = 128
HQ = H // 4

SC_NC = 2
SC_NS = 16
NW = SC_NC * SC_NS
BPW = N * H // NW
CB = 400
NCH = BPW // CB


def _leaky(x):
    return jnp.where(x >= 0, x, LEAK * x)


def _bn_affine(st_ref, g_ref, b_ref):
    mean = st_ref[0:1, :] * (1.0 / N)
    var = st_ref[1:2, :] * (1.0 / N) - mean * mean
    scale = g_ref[...] / jnp.sqrt(var + EPS)
    bias = b_ref[...] - mean * scale
    return scale, bias


def _acc_stats(st_ref, v, width):
    s = jnp.sum(v, axis=0, keepdims=True)
    sq = jnp.sum(v * v, axis=0, keepdims=True)
    blk = jnp.concatenate([s, sq, jnp.zeros((6, width), jnp.float32)], axis=0)

    @pl.when(pl.program_id(0) == 0)
    def _():
        st_ref[...] = jnp.zeros_like(st_ref)

    st_ref[...] += blk


def _k1(x_ref, w_ref, z_ref, st_ref):
    z = jnp.dot(x_ref[...], w_ref[...], preferred_element_type=jnp.float32)
    z_ref[...] = z
    _acc_stats(st_ref, z, CMID)


def _k2(z_ref, p_ref, st_ref, g_ref, b_ref, t_ref):
    scale, bias = _bn_affine(st_ref, g_ref, b_ref)
    x = _leaky(z_ref[...] * scale + bias)
    bm = z_ref.shape[0]
    t_ref[...] = jnp.concatenate(
        [x, p_ref[...], jnp.zeros((bm, 29), jnp.float32)], axis=1)


def _sc_gather(tbl_hbm, idx_hbm, xg_hbm, pr_hbm,
               idx_v, spm_tbl, buf0, buf1, sg0, sg1, sw0, sw1):
    wid = lax.axis_index("s") * SC_NC + lax.axis_index("c")
    sub = lax.axis_index("s")
    pltpu.sync_copy(tbl_hbm.at[pl.ds(sub * (N // SC_NS), N // SC_NS)],
                    spm_tbl.at[pl.ds(sub * (N // SC_NS), N // SC_NS)])
    pltpu.sync_copy(idx_hbm.at[wid], idx_v)
    plsc.subcore_barrier()

    def wrts(c, buf, sem):
        base = wid * BPW + c * CB
        pltpu.async_copy(buf.at[:, 0:32], xg_hbm.at[pl.ds(base, CB)], sem)
        pltpu.async_copy(buf.at[:, 32:64], pr_hbm.at[pl.ds(base, CB)], sem)

    def drain(buf, sem):
        pltpu.make_async_copy(tbl_hbm.at[pl.ds(0, CB)], buf, sem).wait()

    pltpu.async_copy(spm_tbl.at[idx_v.at[0]], buf0, sg0)

    def body(t, _):
        a = 2 * t + 1
        b = 2 * t + 2
        pltpu.async_copy(spm_tbl.at[idx_v.at[a]], buf1, sg1)
        drain(buf0, sg0)
        wrts(2 * t, buf0, sw0)
        drain(buf0, sw0)
        pltpu.async_copy(spm_tbl.at[idx_v.at[b]], buf0, sg0)
        drain(buf1, sg1)
        wrts(a, buf1, sw1)
        drain(buf1, sw1)
        return 0

    lax.fori_loop(0, (NCH - 1) // 2, body, 0)
    drain(buf0, sg0)
    wrts(NCH - 1, buf0, sw0)
    drain(buf0, sw0)


def _k3(xg_ref, pr_ref, q_ref, kpn2_ref, m1_ref, m2_ref, wg1_ref, bg1_ref,
        wg2e_ref, bg2e_ref, wg2o_ref, bg2o_ref, sa_ref, s2_ref,
        out_ref, st_ref):
    bm = out_ref.shape[0]
    xg = xg_ref[...].reshape(bm, HQ, TW)
    qpad = jnp.concatenate(
        [q_ref[...], jnp.zeros((bm, 29), jnp.float32)], axis=1)
    qrep = jnp.tile(qpad, (1, HQ // 2))
    up = pr_ref[...].reshape(bm, HQ, TW) - qrep[:, None, :]
    ukp2 = jnp.dot(up.reshape(bm * HQ, TW), m1_ref[...],
                   preferred_element_type=jnp.float32)
    usq = up * up
    urep = jnp.dot(usq.reshape(bm * HQ, TW), m2_ref[...],
                   preferred_element_type=jnp.float32)
    sq = jnp.maximum(urep + ukp2 + kpn2_ref[...], 0.0)
    infl = jnp.maximum(1.0 - jnp.sqrt(sq) * (1.0 / SIGMA),
                       0.0).reshape(bm, HQ, TW)

    center = xg[:, 0, 0:CMID]
    hmid = _leaky(jnp.dot(center, wg1_ref[...],
                          preferred_element_type=jnp.float32) + bg1_ref[...])
    wv0 = jnp.dot(hmid, wg2e_ref[...],
                  preferred_element_type=jnp.float32) + bg2e_ref[...]
    wv1 = jnp.dot(hmid, wg2o_ref[...],
                  preferred_element_type=jnp.float32) + bg2o_ref[...]
    wvt = jnp.tile(jnp.concatenate([wv0, wv1], axis=1), (1, 4))

    aw = infl * wvt[:, None, :]
    a = jnp.dot(aw.reshape(bm * HQ, TW), sa_ref[...],
                preferred_element_type=jnp.float32).reshape(bm, HQ, TW)
    zs = jnp.sum(xg * a, axis=1)
    outv = jnp.dot(zs, s2_ref[...], preferred_element_type=jnp.float32)
    out_ref[...] = outv
    _acc_stats(st_ref, outv, CMID)


def _k4(y_ref, st2_ref, gk_ref, bk_ref, w2_ref, t_ref, st3_ref):
    scale, bias = _bn_affine(st2_ref, gk_ref, bk_ref)
    l = _leaky(y_ref[...] * scale + bias)
    t = jnp.dot(l, w2_ref[...], preferred_element_type=jnp.float32)
    t_ref[...] = t
    _acc_stats(st3_ref, t, COUT)


def _k5(t_ref, st3_ref, g2_ref, b2_ref, sf_ref, o_ref):
    scale, bias = _bn_affine(st3_ref, g2_ref, b2_ref)
    o_ref[...] = _leaky(t_ref[...] * scale + bias + sf_ref[...])


def kernel(q_pts, s_pts, s_feats, neighbor_indices, kernel_points,
           W1, g1, b1, Wg1, bg1, Wg2, bg2, gk, bk, W2, g2, b2):
    f32 = jnp.float32
    BM1 = 2000
    z, st1 = pl.pallas_call(
        _k1,
        grid=(N // BM1,),
        in_specs=[pl.BlockSpec((BM1, CIN), lambda i: (i, 0)),
                  pl.BlockSpec((CIN, CMID), lambda i: (0, 0))],
        out_specs=[pl.BlockSpec((BM1, CMID), lambda i: (i, 0)),
                   pl.BlockSpec((8, CMID), lambda i: (0, 0))],
        out_shape=[jax.ShapeDtypeStruct((N, CMID), f32),
                   jax.ShapeDtypeStruct((8, CMID), f32)],
    )(s_feats, W1)

    tbl = pl.pallas_call(
        _k2,
        grid=(N // BM1,),
        in_specs=[pl.BlockSpec((BM1, CMID), lambda i: (i, 0)),
                  pl.BlockSpec((BM1, 3), lambda i: (i, 0)),
                  pl.BlockSpec((8, CMID), lambda i: (0, 0)),
                  pl.BlockSpec((1, CMID), lambda i: (0, 0)),
                  pl.BlockSpec((1, CMID), lambda i: (0, 0))],
        out_specs=pl.BlockSpec((BM1, TBW), lambda i: (i, 0)),
        out_shape=jax.ShapeDtypeStruct((N, TBW), f32),
    )(z, s_pts, st1, g1.reshape(1, CMID), b1.reshape(1, CMID))

    gather = pl.kernel(
        _sc_gather,
        out_type=[jax.ShapeDtypeStruct((N * H, 32), f32),
                  jax.ShapeDtypeStruct((N * H, 32), f32)],
        scratch_types=[pltpu.VMEM((NCH, CB), jnp.int32),
                       pltpu.VMEM_SHARED((N, TBW), f32),
                       pltpu.VMEM((CB, TBW), f32),
                       pltpu.VMEM((CB, TBW), f32),
                       pltpu.SemaphoreType.DMA,
                       pltpu.SemaphoreType.DMA,
                       pltpu.SemaphoreType.DMA,
                       pltpu.SemaphoreType.DMA],
        mesh=plsc.VectorSubcoreMesh(core_axis_name="c", subcore_axis_name="s"),
        compiler_params=pltpu.CompilerParams(use_tc_tiling_on_sc=False),
    )
    xg_o, pr_o = gather(tbl, neighbor_indices.reshape(NW, NCH, CB))
    xg = xg_o.reshape(N * H // 4, TW)
    prg = pr_o.reshape(N * H // 4, TW)

    kpad = jnp.concatenate(
        [kernel_points.T, jnp.full((3, 1), 1e6, f32)], axis=1)
    kpb = jnp.concatenate(
        [jnp.tile(kpad, (1, 2)), jnp.zeros((29, 32), f32)], axis=0)
    m1 = jnp.kron(jnp.eye(4, dtype=f32), -2.0 * kpb)
    m2 = jnp.kron(jnp.eye(4, dtype=f32), jnp.ones((32, 32), f32))
    kpn2 = jnp.tile(jnp.sum(kpad * kpad, axis=0).reshape(1, 16), (1, 8))
    Wg2r = jnp.concatenate(
        [Wg2.reshape(HID, K, 2), jnp.zeros((HID, 1, 2), f32)], axis=1)
    bg2r = jnp.concatenate([bg2.reshape(K, 2), jnp.zeros((1, 2), f32)], axis=0)
    sa = jnp.kron(jnp.eye(8, dtype=f32), jnp.ones((16, 16), f32))
    s2 = jnp.kron(jnp.ones((4, 1), f32), jnp.eye(32, dtype=f32))

    BM3 = 1000
    out, st2 = pl.pallas_call(
        _k3,
        grid=(N // BM3,),
        in_specs=[pl.BlockSpec((BM3 * HQ, TW), lambda i: (i, 0)),
                  pl.BlockSpec((BM3 * HQ, TW), lambda i: (i, 0)),
                  pl.BlockSpec((BM3, 3), lambda i: (i, 0)),
                  pl.BlockSpec((1, TW), lambda i: (0, 0)),
                  pl.BlockSpec((TW, TW), lambda i: (0, 0)),
                  pl.BlockSpec((TW, TW), lambda i: (0, 0)),
                  pl.BlockSpec((CMID, HID), lambda i: (0, 0)),
                  pl.BlockSpec((1, HID), lambda i: (0, 0)),
                  pl.BlockSpec((HID, 16), lambda i: (0, 0)),
                  pl.BlockSpec((1, 16), lambda i: (0, 0)),
                  pl.BlockSpec((HID, 16), lambda i: (0, 0)),
                  pl.BlockSpec((1, 16), lambda i: (0, 0)),
                  pl.BlockSpec((TW, TW), lambda i: (0, 0)),
                  pl.BlockSpec((TW, CMID), lambda i: (0, 0))],
        out_specs=[pl.BlockSpec((BM3, CMID), lambda i: (i, 0)),
                   pl.BlockSpec((8, CMID), lambda i: (0, 0))],
        out_shape=[jax.ShapeDtypeStruct((N, CMID), f32),
                   jax.ShapeDtypeStruct((8, CMID), f32)],
    )(xg, prg, q_pts, kpn2, m1, m2, Wg1, bg1.reshape(1, HID),
      Wg2r[:, :, 0], bg2r[:, 0].reshape(1, 16),
      Wg2r[:, :, 1], bg2r[:, 1].reshape(1, 16), sa, s2)

    BM4 = 2000
    t2, st3 = pl.pallas_call(
        _k4,
        grid=(N // BM4,),
        in_specs=[pl.BlockSpec((BM4, CMID), lambda i: (i, 0)),
                  pl.BlockSpec((8, CMID), lambda i: (0, 0)),
                  pl.BlockSpec((1, CMID), lambda i: (0, 0)),
                  pl.BlockSpec((1, CMID), lambda i: (0, 0)),
                  pl.BlockSpec((CMID, COUT), lambda i: (0, 0))],
        out_specs=[pl.BlockSpec((BM4, COUT), lambda i: (i, 0)),
                   pl.BlockSpec((8, COUT), lambda i: (0, 0))],
        out_shape=[jax.ShapeDtypeStruct((N, COUT), f32),
                   jax.ShapeDtypeStruct((8, COUT), f32)],
    )(out, st2, gk.reshape(1, CMID), bk.reshape(1, CMID), W2)

    res = pl.pallas_call(
        _k5,
        grid=(N // BM4,),
        in_specs=[pl.BlockSpec((BM4, COUT), lambda i: (i, 0)),
                  pl.BlockSpec((8, COUT), lambda i: (0, 0)),
                  pl.BlockSpec((1, COUT), lambda i: (0, 0)),
                  pl.BlockSpec((1, COUT), lambda i: (0, 0)),
                  pl.BlockSpec((BM4, COUT), lambda i: (i, 0))],
        out_specs=pl.BlockSpec((BM4, COUT), lambda i: (i, 0)),
        out_shape=jax.ShapeDtypeStruct((N, COUT), f32),
    )(t2, st3, g2.reshape(1, COUT), b2.reshape(1, COUT), s_feats)

    return res


# speedup vs baseline: 2.4444x; 1.4078x over previous
import jax
import jax.numpy as jnp
from jax import lax
from jax.experimental import pallas as pl
from jax.experimental.pallas import tpu as pltpu
from jax.experimental.pallas import tpu_sc as plsc

N = 10000
H = 32
CIN = 128
COUT = 128
CMID = 32
K = 15
CPG = 16
HID = 8
SIGMA = 1.0
LEAK = 0.1
EPS = 1e-5

TBW = 64
TW = 128
HQ = H // 4

SC_NC = 2
SC_NS = 16
NW = SC_NC * SC_NS
BPW = N * H // NW
CB = 400
NCH = BPW // CB


def _leaky(x):
    return jnp.where(x >= 0, x, LEAK * x)


def _bn_affine(st_ref, g_ref, b_ref):
    mean = st_ref[0:1, :] * (1.0 / N)
    var = st_ref[1:2, :] * (1.0 / N) - mean * mean
    scale = g_ref[...] / jnp.sqrt(var + EPS)
    bias = b_ref[...] - mean * scale
    return scale, bias


def _acc_stats(st_ref, v, width):
    s = jnp.sum(v, axis=0, keepdims=True)
    sq = jnp.sum(v * v, axis=0, keepdims=True)
    blk = jnp.concatenate([s, sq, jnp.zeros((6, width), jnp.float32)], axis=0)

    @pl.when(pl.program_id(0) == 0)
    def _():
        st_ref[...] = jnp.zeros_like(st_ref)

    st_ref[...] += blk


def _k1(x_ref, w_ref, z_ref, st_ref):
    z = jnp.dot(x_ref[...], w_ref[...], preferred_element_type=jnp.float32)
    z_ref[...] = z
    _acc_stats(st_ref, z, CMID)


def _k2(z_ref, p_ref, st_ref, g_ref, b_ref, t_ref):
    scale, bias = _bn_affine(st_ref, g_ref, b_ref)
    x = _leaky(z_ref[...] * scale + bias)
    bm = z_ref.shape[0]
    t_ref[...] = jnp.concatenate(
        [x, p_ref[...], jnp.zeros((bm, 29), jnp.float32)], axis=1)


def _sc_gather(tbl_hbm, idx_hbm, xg_hbm, pr_hbm,
               idx_v, spm_x, spm_p, bx0, bp0, bx1, bp1, sg0, sg1, sw0, sw1):
    wid = lax.axis_index("s") * SC_NC + lax.axis_index("c")
    sub = lax.axis_index("s")
    seg = N // SC_NS
    pltpu.sync_copy(tbl_hbm.at[pl.ds(sub * seg, seg), 0:32],
                    spm_x.at[pl.ds(sub * seg, seg)])
    pltpu.sync_copy(tbl_hbm.at[pl.ds(sub * seg, seg), 32:64],
                    spm_p.at[pl.ds(sub * seg, seg)])
    pltpu.sync_copy(idx_hbm.at[wid], idx_v)
    plsc.subcore_barrier()

    def gstart(c, bx, bp, sem):
        pltpu.async_copy(spm_x.at[idx_v.at[c]], bx, sem)
        pltpu.async_copy(spm_p.at[idx_v.at[c]], bp, sem)

    def gdrain(c, bx, bp, sem):
        pltpu.make_async_copy(spm_x.at[idx_v.at[c]], bx, sem).wait()
        pltpu.make_async_copy(spm_p.at[idx_v.at[c]], bp, sem).wait()

    def wstart(c, bx, bp, sem):
        base = wid * BPW + c * CB
        pltpu.async_copy(bx, xg_hbm.at[pl.ds(base, CB)], sem)
        pltpu.async_copy(bp, pr_hbm.at[pl.ds(base, CB)], sem)

    def wdrain(c, bx, bp, sem):
        base = wid * BPW + c * CB
        pltpu.make_async_copy(bx, xg_hbm.at[pl.ds(base, CB)], sem).wait()
        pltpu.make_async_copy(bp, pr_hbm.at[pl.ds(base, CB)], sem).wait()

    gstart(0, bx0, bp0, sg0)

    def body(t, _):
        a = 2 * t + 1
        b = 2 * t + 2
        gstart(a, bx1, bp1, sg1)
        gdrain(2 * t, bx0, bp0, sg0)
        wstart(2 * t, bx0, bp0, sw0)
        wdrain(2 * t, bx0, bp0, sw0)
        gstart(b, bx0, bp0, sg0)
        gdrain(a, bx1, bp1, sg1)
        wstart(a, bx1, bp1, sw1)
        wdrain(a, bx1, bp1, sw1)
        return 0

    lax.fori_loop(0, (NCH - 1) // 2, body, 0)
    gdrain(NCH - 1, bx0, bp0, sg0)
    wstart(NCH - 1, bx0, bp0, sw0)
    wdrain(NCH - 1, bx0, bp0, sw0)


def _k3(xg_ref, pr_ref, q_ref, kpn2_ref, m1_ref, m2_ref, wg1_ref, bg1_ref,
        wg2e_ref, bg2e_ref, wg2o_ref, bg2o_ref, sa_ref, s2_ref,
        out_ref, st_ref):
    bm = out_ref.shape[0]
    xg = xg_ref[...].reshape(bm, HQ, TW)
    qpad = jnp.concatenate(
        [q_ref[...], jnp.zeros((bm, 29), jnp.float32)], axis=1)
    qrep = jnp.tile(qpad, (1, HQ // 2))
    up = pr_ref[...].reshape(bm, HQ, TW) - qrep[:, None, :]
    ukp2 = jnp.dot(up.reshape(bm * HQ, TW), m1_ref[...],
                   preferred_element_type=jnp.float32)
    usq = up * up
    urep = jnp.dot(usq.reshape(bm * HQ, TW), m2_ref[...],
                   preferred_element_type=jnp.float32)
    sq = jnp.maximum(urep + ukp2 + kpn2_ref[...], 0.0)
    infl = jnp.maximum(1.0 - jnp.sqrt(sq) * (1.0 / SIGMA),
                       0.0).reshape(bm, HQ, TW)

    center = xg[:, 0, 0:CMID]
    hmid = _leaky(jnp.dot(center, wg1_ref[...],
                          preferred_element_type=jnp.float32) + bg1_ref[...])
    wv0 = jnp.dot(hmid, wg2e_ref[...],
                  preferred_element_type=jnp.float32) + bg2e_ref[...]
    wv1 = jnp.dot(hmid, wg2o_ref[...],
                  preferred_element_type=jnp.float32) + bg2o_ref[...]
    wvt = jnp.tile(jnp.concatenate([wv0, wv1], axis=1), (1, 4))

    aw = infl * wvt[:, None, :]
    a = jnp.dot(aw.reshape(bm * HQ, TW), sa_ref[...],
                preferred_element_type=jnp.float32).reshape(bm, HQ, TW)
    zs = jnp.sum(xg * a, axis=1)
    outv = jnp.dot(zs, s2_ref[...], preferred_element_type=jnp.float32)
    out_ref[...] = outv
    _acc_stats(st_ref, outv, CMID)


def _k4(y_ref, st2_ref, gk_ref, bk_ref, w2_ref, t_ref, st3_ref):
    scale, bias = _bn_affine(st2_ref, gk_ref, bk_ref)
    l = _leaky(y_ref[...] * scale + bias)
    t = jnp.dot(l, w2_ref[...], preferred_element_type=jnp.float32)
    t_ref[...] = t
    _acc_stats(st3_ref, t, COUT)


def _k5(t_ref, st3_ref, g2_ref, b2_ref, sf_ref, o_ref):
    scale, bias = _bn_affine(st3_ref, g2_ref, b2_ref)
    o_ref[...] = _leaky(t_ref[...] * scale + bias + sf_ref[...])


def kernel(q_pts, s_pts, s_feats, neighbor_indices, kernel_points,
           W1, g1, b1, Wg1, bg1, Wg2, bg2, gk, bk, W2, g2, b2):
    f32 = jnp.float32
    BM1 = 2000
    z, st1 = pl.pallas_call(
        _k1,
        grid=(N // BM1,),
        in_specs=[pl.BlockSpec((BM1, CIN), lambda i: (i, 0)),
                  pl.BlockSpec((CIN, CMID), lambda i: (0, 0))],
        out_specs=[pl.BlockSpec((BM1, CMID), lambda i: (i, 0)),
                   pl.BlockSpec((8, CMID), lambda i: (0, 0))],
        out_shape=[jax.ShapeDtypeStruct((N, CMID), f32),
                   jax.ShapeDtypeStruct((8, CMID), f32)],
    )(s_feats, W1)

    tbl = pl.pallas_call(
        _k2,
        grid=(N // BM1,),
        in_specs=[pl.BlockSpec((BM1, CMID), lambda i: (i, 0)),
                  pl.BlockSpec((BM1, 3), lambda i: (i, 0)),
                  pl.BlockSpec((8, CMID), lambda i: (0, 0)),
                  pl.BlockSpec((1, CMID), lambda i: (0, 0)),
                  pl.BlockSpec((1, CMID), lambda i: (0, 0))],
        out_specs=pl.BlockSpec((BM1, TBW), lambda i: (i, 0)),
        out_shape=jax.ShapeDtypeStruct((N, TBW), f32),
    )(z, s_pts, st1, g1.reshape(1, CMID), b1.reshape(1, CMID))

    gather = pl.kernel(
        _sc_gather,
        out_type=[jax.ShapeDtypeStruct((N * H, 32), f32),
                  jax.ShapeDtypeStruct((N * H, 32), f32)],
        scratch_types=[pltpu.VMEM((NCH, CB), jnp.int32),
                       pltpu.VMEM_SHARED((N, 32), f32),
                       pltpu.VMEM_SHARED((N, 32), f32),
                       pltpu.VMEM((CB, 32), f32),
                       pltpu.VMEM((CB, 32), f32),
                       pltpu.VMEM((CB, 32), f32),
                       pltpu.VMEM((CB, 32), f32),
                       pltpu.SemaphoreType.DMA,
                       pltpu.SemaphoreType.DMA,
                       pltpu.SemaphoreType.DMA,
                       pltpu.SemaphoreType.DMA],
        mesh=plsc.VectorSubcoreMesh(core_axis_name="c", subcore_axis_name="s"),
        compiler_params=pltpu.CompilerParams(use_tc_tiling_on_sc=False),
    )
    xg_o, pr_o = gather(tbl, neighbor_indices.reshape(NW, NCH, CB))
    xg = xg_o.reshape(N * H // 4, TW)
    prg = pr_o.reshape(N * H // 4, TW)

    kpad = jnp.concatenate(
        [kernel_points.T, jnp.full((3, 1), 1e6, f32)], axis=1)
    kpb = jnp.concatenate(
        [jnp.tile(kpad, (1, 2)), jnp.zeros((29, 32), f32)], axis=0)
    m1 = jnp.kron(jnp.eye(4, dtype=f32), -2.0 * kpb)
    m2 = jnp.kron(jnp.eye(4, dtype=f32), jnp.ones((32, 32), f32))
    kpn2 = jnp.tile(jnp.sum(kpad * kpad, axis=0).reshape(1, 16), (1, 8))
    Wg2r = jnp.concatenate(
        [Wg2.reshape(HID, K, 2), jnp.zeros((HID, 1, 2), f32)], axis=1)
    bg2r = jnp.concatenate([bg2.reshape(K, 2), jnp.zeros((1, 2), f32)], axis=0)
    sa = jnp.kron(jnp.eye(8, dtype=f32), jnp.ones((16, 16), f32))
    s2 = jnp.kron(jnp.ones((4, 1), f32), jnp.eye(32, dtype=f32))

    BM3 = 1000
    out, st2 = pl.pallas_call(
        _k3,
        grid=(N // BM3,),
        in_specs=[pl.BlockSpec((BM3 * HQ, TW), lambda i: (i, 0)),
                  pl.BlockSpec((BM3 * HQ, TW), lambda i: (i, 0)),
                  pl.BlockSpec((BM3, 3), lambda i: (i, 0)),
                  pl.BlockSpec((1, TW), lambda i: (0, 0)),
                  pl.BlockSpec((TW, TW), lambda i: (0, 0)),
                  pl.BlockSpec((TW, TW), lambda i: (0, 0)),
                  pl.BlockSpec((CMID, HID), lambda i: (0, 0)),
                  pl.BlockSpec((1, HID), lambda i: (0, 0)),
                  pl.BlockSpec((HID, 16), lambda i: (0, 0)),
                  pl.BlockSpec((1, 16), lambda i: (0, 0)),
                  pl.BlockSpec((HID, 16), lambda i: (0, 0)),
                  pl.BlockSpec((1, 16), lambda i: (0, 0)),
                  pl.BlockSpec((TW, TW), lambda i: (0, 0)),
                  pl.BlockSpec((TW, CMID), lambda i: (0, 0))],
        out_specs=[pl.BlockSpec((BM3, CMID), lambda i: (i, 0)),
                   pl.BlockSpec((8, CMID), lambda i: (0, 0))],
        out_shape=[jax.ShapeDtypeStruct((N, CMID), f32),
                   jax.ShapeDtypeStruct((8, CMID), f32)],
    )(xg, prg, q_pts, kpn2, m1, m2, Wg1, bg1.reshape(1, HID),
      Wg2r[:, :, 0], bg2r[:, 0].reshape(1, 16),
      Wg2r[:, :, 1], bg2r[:, 1].reshape(1, 16), sa, s2)

    BM4 = 2000
    t2, st3 = pl.pallas_call(
        _k4,
        grid=(N // BM4,),
        in_specs=[pl.BlockSpec((BM4, CMID), lambda i: (i, 0)),
                  pl.BlockSpec((8, CMID), lambda i: (0, 0)),
                  pl.BlockSpec((1, CMID), lambda i: (0, 0)),
                  pl.BlockSpec((1, CMID), lambda i: (0, 0)),
                  pl.BlockSpec((CMID, COUT), lambda i: (0, 0))],
        out_specs=[pl.BlockSpec((BM4, COUT), lambda i: (i, 0)),
                   pl.BlockSpec((8, COUT), lambda i: (0, 0))],
        out_shape=[jax.ShapeDtypeStruct((N, COUT), f32),
                   jax.ShapeDtypeStruct((8, COUT), f32)],
    )(out, st2, gk.reshape(1, CMID), bk.reshape(1, CMID), W2)

    res = pl.pallas_call(
        _k5,
        grid=(N // BM4,),
        in_specs=[pl.BlockSpec((BM4, COUT), lambda i: (i, 0)),
                  pl.BlockSpec((8, COUT), lambda i: (0, 0)),
                  pl.BlockSpec((1, COUT), lambda i: (0, 0)),
                  pl.BlockSpec((1, COUT), lambda i: (0, 0)),
                  pl.BlockSpec((BM4, COUT), lambda i: (i, 0))],
        out_specs=pl.BlockSpec((BM4, COUT), lambda i: (i, 0)),
        out_shape=jax.ShapeDtypeStruct((N, COUT), f32),
    )(t2, st3, g2.reshape(1, COUT), b2.reshape(1, COUT), s_feats)

    return res
